# Initial kernel scaffold; baseline (speedup 1.0000x reference)
import jax
import jax.numpy as jnp
from jax.experimental import pallas as pl


def kernel(x, table):
    raise NotImplementedError("write your pallas kernel here")


# speedup vs baseline: 1.1582x; 1.1582x over previous
import functools
import math

import jax
import jax.numpy as jnp
from jax import lax
from jax.experimental import pallas as pl
from jax.experimental.pallas import tpu as pltpu
from jax.experimental.pallas import tpu_sc as plsc

D_MODEL = 768
SCALE = math.sqrt(D_MODEL)

NUM_CORES = 2
NUM_SUBCORES = 16
NW = NUM_CORES * NUM_SUBCORES
BATCH = 4 * 4096
BPW = BATCH // NW
CHUNK = 64
NCHUNK = BPW // CHUNK
LANES = 16

_mesh = plsc.VectorSubcoreMesh(core_axis_name="c", subcore_axis_name="s")


@functools.partial(
    pl.kernel,
    mesh=_mesh,
    out_type=jax.ShapeDtypeStruct((BATCH, D_MODEL), jnp.float32),
    scratch_types=[
        pltpu.VMEM((NCHUNK, CHUNK), jnp.int32),
        pltpu.VMEM((CHUNK, D_MODEL), jnp.float32),
        pltpu.SemaphoreType.DMA,
    ],
)
def _emb_lookup(idx_hbm, table_hbm, out_hbm, idx_v, buf_v, sem):
    wid = lax.axis_index("s") * NUM_CORES + lax.axis_index("c")
    base = wid * BPW
    pltpu.sync_copy(idx_hbm.at[wid], idx_v)

    def scale_row(r, carry):
        for j in range(D_MODEL // LANES):
            sl = pl.ds(j * LANES, LANES)
            buf_v[r, sl] = buf_v[r, sl] * SCALE
        return carry

    for g in range(NCHUNK):
        pltpu.async_copy(table_hbm.at[idx_v.at[g]], buf_v, sem).wait()
        lax.fori_loop(0, CHUNK, scale_row, 0)
        pltpu.sync_copy(buf_v, out_hbm.at[pl.ds(base + g * CHUNK, CHUNK)])


def kernel(x, table):
    ids = x.reshape(-1).astype(jnp.int32).reshape(NW, NCHUNK, CHUNK)
    out = _emb_lookup(ids, table)
    return out.reshape(x.shape + (D_MODEL,))


# speedup vs baseline: 1.2453x; 1.0753x over previous
import functools
import math

import jax
import jax.numpy as jnp
from jax import lax
from jax.experimental import pallas as pl
from jax.experimental.pallas import tpu as pltpu
from jax.experimental.pallas import tpu_sc as plsc

D_MODEL = 768
SCALE = math.sqrt(D_MODEL)

NUM_CORES = 2
NUM_SUBCORES = 16
NW = NUM_CORES * NUM_SUBCORES
BATCH = 4 * 4096
BPW = BATCH // NW
CHUNK = 64
NCHUNK = BPW // CHUNK
NBUF = 2
LANES = 16

_mesh = plsc.VectorSubcoreMesh(core_axis_name="c", subcore_axis_name="s")


@functools.partial(
    pl.kernel,
    mesh=_mesh,
    out_type=jax.ShapeDtypeStruct((BATCH, D_MODEL), jnp.float32),
    scratch_types=[
        pltpu.VMEM((NCHUNK, CHUNK), jnp.int32),
        *[pltpu.VMEM((CHUNK, D_MODEL), jnp.float32) for _ in range(NBUF)],
        *[pltpu.SemaphoreType.DMA for _ in range(2 * NBUF)],
    ],
)
def _emb_lookup(idx_hbm, table_hbm, out_hbm, idx_v, *bufs_and_sems):
    bufs = bufs_and_sems[:NBUF]
    gsems = bufs_and_sems[NBUF:2 * NBUF]
    ssems = bufs_and_sems[2 * NBUF:]
    wid = lax.axis_index("s") * NUM_CORES + lax.axis_index("c")
    base = wid * BPW
    pltpu.sync_copy(idx_hbm.at[wid], idx_v)

    def gather(g):
        return pltpu.async_copy(
            table_hbm.at[idx_v.at[g]], bufs[g % NBUF], gsems[g % NBUF])

    def store(g):
        return pltpu.async_copy(
            bufs[g % NBUF], out_hbm.at[pl.ds(base + g * CHUNK, CHUNK)],
            ssems[g % NBUF])

    def scale(buf):
        @plsc.parallel_loop(0, CHUNK, step=1, unroll=2)
        def _row(r):
            for j in range(D_MODEL // LANES):
                sl = pl.ds(j * LANES, LANES)
                buf[r, sl] = buf[r, sl] * SCALE

    h_g = {}
    h_s = {}
    for g in range(min(NBUF, NCHUNK)):
        h_g[g] = gather(g)
    for g in range(NCHUNK):
        h_g[g].wait()
        scale(bufs[g % NBUF])
        h_s[g] = store(g)
        pg = g + NBUF - 1
        if g >= 1 and pg < NCHUNK:
            h_s[g - 1].wait()
            h_g[pg] = gather(pg)
    for g in range(max(0, NCHUNK - NBUF + 1), NCHUNK):
        h_s[g].wait()


def kernel(x, table):
    ids = x.reshape(-1).astype(jnp.int32).reshape(NW, NCHUNK, CHUNK)
    out = _emb_lookup(ids, table)
    return out.reshape(x.shape + (D_MODEL,))


# speedup vs baseline: 1.3405x; 1.0764x over previous
import functools
import math

import jax
import jax.numpy as jnp
from jax import lax
from jax.experimental import pallas as pl
from jax.experimental.pallas import tpu as pltpu
from jax.experimental.pallas import tpu_sc as plsc

D_MODEL = 768
SCALE = math.sqrt(D_MODEL)

NUM_CORES = 2
NUM_SUBCORES = 16
NW = NUM_CORES * NUM_SUBCORES
BATCH = 4 * 4096
BPW = BATCH // NW
CHUNK = 32
NCHUNK = BPW // CHUNK
NBUF = 4
LANES = 16

_mesh = plsc.VectorSubcoreMesh(core_axis_name="c", subcore_axis_name="s")


@functools.partial(
    pl.kernel,
    mesh=_mesh,
    out_type=jax.ShapeDtypeStruct((BATCH, D_MODEL), jnp.float32),
    scratch_types=[
        pltpu.VMEM((NCHUNK, CHUNK), jnp.int32),
        *[pltpu.VMEM((CHUNK, D_MODEL), jnp.float32) for _ in range(NBUF)],
        *[pltpu.SemaphoreType.DMA for _ in range(2 * NBUF)],
    ],
)
def _emb_lookup(idx_hbm, table_hbm, out_hbm, idx_v, *bufs_and_sems):
    bufs = bufs_and_sems[:NBUF]
    gsems = bufs_and_sems[NBUF:2 * NBUF]
    ssems = bufs_and_sems[2 * NBUF:]
    wid = lax.axis_index("s") * NUM_CORES + lax.axis_index("c")
    base = wid * BPW
    pltpu.sync_copy(idx_hbm.at[wid], idx_v)

    def gather(g):
        return pltpu.async_copy(
            table_hbm.at[idx_v.at[g]], bufs[g % NBUF], gsems[g % NBUF])

    def store(g):
        return pltpu.async_copy(
            bufs[g % NBUF], out_hbm.at[pl.ds(base + g * CHUNK, CHUNK)],
            ssems[g % NBUF])

    def scale(buf):
        @plsc.parallel_loop(0, CHUNK, step=1, unroll=2)
        def _row(r):
            for j in range(D_MODEL // LANES):
                sl = pl.ds(j * LANES, LANES)
                buf[r, sl] = buf[r, sl] * SCALE

    h_g = {}
    h_s = {}
    for g in range(min(NBUF, NCHUNK)):
        h_g[g] = gather(g)
    for g in range(NCHUNK):
        h_g[g].wait()
        scale(bufs[g % NBUF])
        h_s[g] = store(g)
        pg = g + NBUF - 1
        if g >= 1 and pg < NCHUNK:
            h_s[g - 1].wait()
            h_g[pg] = gather(pg)
    for g in range(max(0, NCHUNK - NBUF + 1), NCHUNK):
        h_s[g].wait()


def kernel(x, table):
    ids = x.reshape(-1).astype(jnp.int32).reshape(NW, NCHUNK, CHUNK)
    out = _emb_lookup(ids, table)
    return out.reshape(x.shape + (D_MODEL,))


# speedup vs baseline: 1.3405x; 1.0000x over previous
import functools
import math

import jax
import jax.numpy as jnp
from jax import lax
from jax.experimental import pallas as pl
from jax.experimental.pallas import tpu as pltpu
from jax.experimental.pallas import tpu_sc as plsc

D_MODEL = 768
SCALE = math.sqrt(D_MODEL)

NUM_CORES = 2
NUM_SUBCORES = 16
NW = NUM_CORES * NUM_SUBCORES
BATCH = 4 * 4096
BPW = BATCH // NW
CHUNK = 32
NCHUNK = BPW // CHUNK
NBUF = 5
LANES = 16

_mesh = plsc.VectorSubcoreMesh(core_axis_name="c", subcore_axis_name="s")


@functools.partial(
    pl.kernel,
    mesh=_mesh,
    out_type=jax.ShapeDtypeStruct((BATCH, D_MODEL), jnp.float32),
    scratch_types=[
        pltpu.VMEM((BPW,), jnp.int32),
        *[pltpu.VMEM((CHUNK, D_MODEL), jnp.float32) for _ in range(NBUF)],
        *[pltpu.SemaphoreType.DMA for _ in range(2 * NBUF)],
    ],
)
def _emb_lookup(idx_hbm, table_hbm, out_hbm, idx_v, *bufs_and_sems):
    bufs = bufs_and_sems[:NBUF]
    gsems = bufs_and_sems[NBUF:2 * NBUF]
    ssems = bufs_and_sems[2 * NBUF:]
    wid = lax.axis_index("s") * NUM_CORES + lax.axis_index("c")
    base = wid * BPW
    pltpu.sync_copy(idx_hbm.at[pl.ds(base, BPW)], idx_v)

    def gather(g):
        return pltpu.async_copy(
            table_hbm.at[idx_v.at[pl.ds(g * CHUNK, CHUNK)]],
            bufs[g % NBUF], gsems[g % NBUF])

    def store(g):
        return pltpu.async_copy(
            bufs[g % NBUF], out_hbm.at[pl.ds(base + g * CHUNK, CHUNK)],
            ssems[g % NBUF])

    def scale(buf):
        @plsc.parallel_loop(0, CHUNK, step=1, unroll=4)
        def _row(r):
            for j in range(D_MODEL // LANES):
                sl = pl.ds(j * LANES, LANES)
                buf[r, sl] = buf[r, sl] * SCALE

    h_g = {}
    h_s = {}
    for g in range(min(NBUF, NCHUNK)):
        h_g[g] = gather(g)
    for g in range(NCHUNK):
        h_g[g].wait()
        scale(bufs[g % NBUF])
        h_s[g] = store(g)
        pg = g + NBUF - 1
        if g >= 1 and pg < NCHUNK:
            h_s[g - 1].wait()
            h_g[pg] = gather(pg)
    for g in range(max(0, NCHUNK - NBUF + 1), NCHUNK):
        h_s[g].wait()


def kernel(x, table):
    ids = x.reshape(-1).astype(jnp.int32)
    out = _emb_lookup(ids, table)
    return out.reshape(x.shape + (D_MODEL,))


# speedup vs baseline: 1.3437x; 1.0024x over previous
import functools
import math

import jax
import jax.numpy as jnp
from jax import lax
from jax.experimental import pallas as pl
from jax.experimental.pallas import tpu as pltpu
from jax.experimental.pallas import tpu_sc as plsc

D_MODEL = 768
SCALE = math.sqrt(D_MODEL)

NUM_CORES = 2
NUM_SUBCORES = 16
NW = NUM_CORES * NUM_SUBCORES
ROWS = 4
COLS = 4096
BATCH = ROWS * COLS
BPW = BATCH // NW
WPR = COLS // BPW
CHUNK = 32
NCHUNK = BPW // CHUNK
NBUF = 4
LANES = 16

_mesh = plsc.VectorSubcoreMesh(core_axis_name="c", subcore_axis_name="s")


@functools.partial(
    pl.kernel,
    mesh=_mesh,
    out_type=jax.ShapeDtypeStruct((ROWS, COLS, D_MODEL), jnp.float32),
    scratch_types=[
        pltpu.VMEM((BPW,), jnp.int32),
        *[pltpu.VMEM((CHUNK, D_MODEL), jnp.float32) for _ in range(NBUF)],
        *[pltpu.SemaphoreType.DMA for _ in range(2 * NBUF)],
    ],
)
def _emb_lookup(idx_hbm, table_hbm, out_hbm, idx_v, *bufs_and_sems):
    bufs = bufs_and_sems[:NBUF]
    gsems = bufs_and_sems[NBUF:2 * NBUF]
    ssems = bufs_and_sems[2 * NBUF:]
    wid = lax.axis_index("s") * NUM_CORES + lax.axis_index("c")
    row = wid // WPR
    col0 = (wid % WPR) * BPW
    pltpu.sync_copy(idx_hbm.at[row, pl.ds(col0, BPW)], idx_v)

    def gather(g):
        return pltpu.async_copy(
            table_hbm.at[idx_v.at[pl.ds(g * CHUNK, CHUNK)]],
            bufs[g % NBUF], gsems[g % NBUF])

    def store(g):
        return pltpu.async_copy(
            bufs[g % NBUF],
            out_hbm.at[row, pl.ds(col0 + g * CHUNK, CHUNK)],
            ssems[g % NBUF])

    def scale(buf):
        @plsc.parallel_loop(0, CHUNK, step=1, unroll=2)
        def _row(r):
            for j in range(D_MODEL // LANES):
                sl = pl.ds(j * LANES, LANES)
                buf[r, sl] = buf[r, sl] * SCALE

    h_g = {}
    h_s = {}
    for g in range(min(NBUF, NCHUNK)):
        h_g[g] = gather(g)
    for g in range(NCHUNK):
        h_g[g].wait()
        scale(bufs[g % NBUF])
        h_s[g] = store(g)
        pg = g + NBUF - 1
        if g >= 1 and pg < NCHUNK:
            h_s[g - 1].wait()
            h_g[pg] = gather(pg)
    for g in range(max(0, NCHUNK - NBUF + 1), NCHUNK):
        h_s[g].wait()


def kernel(x, table):
    return _emb_lookup(x.astype(jnp.int32), table)


# speedup vs baseline: 1.5280x; 1.1371x over previous
import functools
import math

import jax
import jax.numpy as jnp
from jax import lax
from jax.experimental import pallas as pl
from jax.experimental.pallas import tpu as pltpu
from jax.experimental.pallas import tpu_sc as plsc

D_MODEL = 768
SCALE = math.sqrt(D_MODEL)

NUM_CORES = 2
NUM_SUBCORES = 16
NW = NUM_CORES * NUM_SUBCORES
ROWS = 4
COLS = 4096
BATCH = ROWS * COLS
BPW = BATCH // NW
WPR = COLS // BPW
CHUNK = 32
NCHUNK = BPW // CHUNK
NBUF = 4
NRING = NCHUNK // NBUF
LANES = 16

_mesh = plsc.VectorSubcoreMesh(core_axis_name="c", subcore_axis_name="s")


@functools.partial(
    pl.kernel,
    mesh=_mesh,
    out_type=jax.ShapeDtypeStruct((ROWS, COLS, D_MODEL), jnp.float32),
    scratch_types=[
        pltpu.VMEM((BPW,), jnp.int32),
        *[pltpu.VMEM((CHUNK, D_MODEL), jnp.float32) for _ in range(NBUF)],
        *[pltpu.SemaphoreType.DMA for _ in range(2 * NBUF)],
    ],
)
def _emb_lookup(idx_hbm, table_hbm, out_hbm, idx_v, *bufs_and_sems):
    bufs = bufs_and_sems[:NBUF]
    gsems = bufs_and_sems[NBUF:2 * NBUF]
    ssems = bufs_and_sems[2 * NBUF:]
    wid = lax.axis_index("s") * NUM_CORES + lax.axis_index("c")
    row = wid // WPR
    col0 = (wid % WPR) * BPW
    pltpu.sync_copy(idx_hbm.at[row, pl.ds(col0, BPW)], idx_v)

    def gather_desc(g, b):
        return pltpu.make_async_copy(
            table_hbm.at[idx_v.at[pl.ds(g * CHUNK, CHUNK)]],
            bufs[b], gsems[b])

    def store_desc(g, b):
        return pltpu.make_async_copy(
            bufs[b], out_hbm.at[row, pl.ds(col0 + g * CHUNK, CHUNK)],
            ssems[b])

    def scale(buf):
        @plsc.parallel_loop(0, CHUNK, step=1, unroll=2)
        def _row(r):
            for j in range(D_MODEL // LANES):
                sl = pl.ds(j * LANES, LANES)
                buf[r, sl] = buf[r, sl] * SCALE

    for g in range(NBUF):
        gather_desc(g, g).start()

    def ring(t, carry):
        for b in range(NBUF):
            g = t * NBUF + b
            pg = g + NBUF - 1
            pb = (b - 1) % NBUF

            @pl.when(jnp.logical_and(g >= 1, pg < NCHUNK))
            def _():
                store_desc(g - 1, pb).wait()
                gather_desc(pg, pb).start()

            gather_desc(g, b).wait()
            scale(bufs[b])
            store_desc(g, b).start()
        return carry

    lax.fori_loop(0, NRING, ring, 0)
    for g in range(NCHUNK - NBUF + 1, NCHUNK):
        store_desc(g, g % NBUF).wait()


def kernel(x, table):
    return _emb_lookup(x.astype(jnp.int32), table)
